# baseline (speedup 1.0000x reference)
import jax
import jax.numpy as jnp
from jax import lax
from jax.experimental import pallas as pl
from jax.experimental.pallas import tpu as pltpu
from jax.experimental.pallas import tpu_sc as plsc

N_REAL = 10000
N_PAD = 10240
D = 128
NC, NS = 2, 16
NW = NC * NS
ROWS_PER_TILE = N_PAD // NS
E_PER_TILE = 10000
SLABS = 78
SLAB_W = 128
TAIL_W = E_PER_TILE - SLABS * SLAB_W
ROW_BLK = 2000


def _mesh():
    return plsc.VectorSubcoreMesh(core_axis_name="c", subcore_axis_name="s")


def _mm_body(x_ref, w_ref, xw_ref):
    xw_ref[...] = jnp.dot(x_ref[...], w_ref[...],
                          preferred_element_type=jnp.float32)


@jax.jit
def _mm_call(x, W):
    return pl.pallas_call(
        _mm_body,
        grid=(N_REAL // ROW_BLK,),
        in_specs=[
            pl.BlockSpec((ROW_BLK, D), lambda i: (i, 0)),
            pl.BlockSpec((D, D), lambda i: (0, 0)),
        ],
        out_specs=pl.BlockSpec((ROW_BLK, D), lambda i: (i, 0)),
        out_shape=jax.ShapeDtypeStruct((N_REAL, D), jnp.float32),
    )(x, W)


def _deg_body(dst_hbm, dtl_hbm, deg_out, idx_v, tidx_v, ones_v, zeros_v,
              deg_sh, sem):
    c = lax.axis_index("c")
    s = lax.axis_index("s")
    w = c * NS + s
    for i in range(SLAB_W // 16):
        ones_v[pl.ds(i * 16, 16)] = jnp.full((16,), 1.0, jnp.float32)
    for i in range(ROWS_PER_TILE // 16):
        zeros_v[pl.ds(i * 16, 16)] = jnp.zeros((16,), jnp.float32)
    pltpu.sync_copy(zeros_v, deg_sh.at[pl.ds(s * ROWS_PER_TILE, ROWS_PER_TILE)])
    pltpu.async_copy(dst_hbm.at[w], idx_v, sem).wait()
    pltpu.async_copy(dtl_hbm.at[w], tidx_v, sem).wait()
    plsc.subcore_barrier()

    @pl.loop(0, SLABS)
    def _(j):
        pltpu.sync_copy(ones_v, deg_sh.at[idx_v.at[j]], add=True)

    pltpu.sync_copy(ones_v.at[pl.ds(0, TAIL_W)], deg_sh.at[tidx_v.at[0]],
                    add=True)
    plsc.subcore_barrier()
    pltpu.sync_copy(deg_sh.at[pl.ds(s * ROWS_PER_TILE, ROWS_PER_TILE)],
                    deg_out.at[c, pl.ds(s * ROWS_PER_TILE, ROWS_PER_TILE)])


@jax.jit
def _deg_call(dst_slab, dst_tail):
    f = pl.kernel(
        _deg_body,
        out_type=jax.ShapeDtypeStruct((NC, N_PAD), jnp.float32),
        mesh=_mesh(),
        scratch_types=[
            pltpu.VMEM((SLABS, SLAB_W), jnp.int32),
            pltpu.VMEM((1, TAIL_W), jnp.int32),
            pltpu.VMEM((SLAB_W,), jnp.float32),
            pltpu.VMEM((ROWS_PER_TILE,), jnp.float32),
            pltpu.VMEM_SHARED((N_PAD,), jnp.float32),
            pltpu.SemaphoreType.DMA,
        ],
    )
    return f(dst_slab, dst_tail)


def _scale_body(xw_ref, deg_ref, y_ref, dinv_ref):
    d = deg_ref[0] + deg_ref[1] + 1.0
    dinv = lax.rsqrt(d)
    y_ref[...] = xw_ref[...] * dinv
    dinv_ref[...] = dinv


@jax.jit
def _scale_call(xw, deg3):
    return pl.pallas_call(
        _scale_body,
        grid=(N_REAL // ROW_BLK,),
        in_specs=[
            pl.BlockSpec((ROW_BLK, D), lambda i: (i, 0)),
            pl.BlockSpec((NC, ROW_BLK, 1), lambda i: (0, i, 0)),
        ],
        out_specs=[
            pl.BlockSpec((ROW_BLK, D), lambda i: (i, 0)),
            pl.BlockSpec((ROW_BLK, 1), lambda i: (i, 0)),
        ],
        out_shape=[
            jax.ShapeDtypeStruct((N_REAL, D), jnp.float32),
            jax.ShapeDtypeStruct((N_REAL, 1), jnp.float32),
        ],
    )(xw, deg3)


def _msg_body(y_hbm, src_hbm, stl_hbm, dst_hbm, dtl_hbm, acc_out,
              src_v, stidx_v, dst_v, dtidx_v, buf0, acc_sh, sem, sem0):
    c = lax.axis_index("c")
    s = lax.axis_index("s")
    w = c * NS + s
    base = s * ROWS_PER_TILE

    @pl.loop(0, SLAB_W)
    def _(i):
        for k in range(D // 16):
            buf0[i, pl.ds(k * 16, 16)] = jnp.zeros((16,), jnp.float32)

    for t in range(ROWS_PER_TILE // SLAB_W):
        pltpu.sync_copy(buf0, acc_sh.at[pl.ds(base + t * SLAB_W, SLAB_W)])

    @pl.when(c == 0)
    def _():
        @pl.when(s < NS - 1)
        def _():
            pltpu.sync_copy(y_hbm.at[pl.ds(base, ROWS_PER_TILE)],
                            acc_sh.at[pl.ds(base, ROWS_PER_TILE)])

        @pl.when(s == NS - 1)
        def _():
            n_last = N_REAL - (NS - 1) * ROWS_PER_TILE
            pltpu.sync_copy(y_hbm.at[pl.ds(base, n_last)],
                            acc_sh.at[pl.ds(base, n_last)])

    pltpu.async_copy(src_hbm.at[w], src_v, sem).wait()
    pltpu.async_copy(stl_hbm.at[w], stidx_v, sem).wait()
    pltpu.async_copy(dst_hbm.at[w], dst_v, sem).wait()
    pltpu.async_copy(dtl_hbm.at[w], dtidx_v, sem).wait()
    plsc.subcore_barrier()

    @pl.loop(0, SLABS)
    def _(j):
        pltpu.async_copy(y_hbm.at[src_v.at[j]], buf0, sem0).wait()
        pltpu.sync_copy(buf0, acc_sh.at[dst_v.at[j]], add=True)

    pltpu.async_copy(y_hbm.at[stidx_v.at[0]], buf0.at[pl.ds(0, TAIL_W)],
                     sem0).wait()
    pltpu.sync_copy(buf0.at[pl.ds(0, TAIL_W)], acc_sh.at[dtidx_v.at[0]],
                    add=True)

    plsc.subcore_barrier()
    pltpu.sync_copy(acc_sh.at[pl.ds(base, ROWS_PER_TILE)],
                    acc_out.at[c, pl.ds(base, ROWS_PER_TILE)])


@jax.jit
def _msg_call(y, src_slab, src_tail, dst_slab, dst_tail):
    f = pl.kernel(
        _msg_body,
        out_type=jax.ShapeDtypeStruct((NC, N_PAD, D), jnp.float32),
        mesh=_mesh(),
        scratch_types=[
            pltpu.VMEM((SLABS, SLAB_W), jnp.int32),
            pltpu.VMEM((1, TAIL_W), jnp.int32),
            pltpu.VMEM((SLABS, SLAB_W), jnp.int32),
            pltpu.VMEM((1, TAIL_W), jnp.int32),
            pltpu.VMEM((SLAB_W, D), jnp.float32),
            pltpu.VMEM_SHARED((N_PAD, D), jnp.float32),
            pltpu.SemaphoreType.DMA,
            pltpu.SemaphoreType.DMA,
        ],
    )
    return f(y, src_slab, src_tail, dst_slab, dst_tail)


def _out_body(acc_ref, dinv_ref, o_ref):
    o_ref[...] = jnp.maximum((acc_ref[0] + acc_ref[1]) * dinv_ref[...], 0.0)


@jax.jit
def _out_call(acc, dinv):
    return pl.pallas_call(
        _out_body,
        grid=(N_REAL // ROW_BLK,),
        in_specs=[
            pl.BlockSpec((NC, ROW_BLK, D), lambda i: (0, i, 0)),
            pl.BlockSpec((ROW_BLK, 1), lambda i: (i, 0)),
        ],
        out_specs=pl.BlockSpec((ROW_BLK, D), lambda i: (i, 0)),
        out_shape=jax.ShapeDtypeStruct((N_REAL, D), jnp.float32),
    )(acc, dinv)


def kernel(x, edge_index, W):
    src = edge_index[0].astype(jnp.int32)
    dst = edge_index[1].astype(jnp.int32)
    n_main = NW * SLABS * SLAB_W
    src_slab = src[:n_main].reshape(NW, SLABS, SLAB_W)
    dst_slab = dst[:n_main].reshape(NW, SLABS, SLAB_W)
    src_tail = src[n_main:].reshape(NW, 1, TAIL_W)
    dst_tail = dst[n_main:].reshape(NW, 1, TAIL_W)

    xw = _mm_call(x, W)
    deg_part = _deg_call(dst_slab, dst_tail)
    deg3 = deg_part.reshape(NC, N_PAD, 1)
    y, dinv = _scale_call(xw, deg3)
    acc = _msg_call(y, src_slab, src_tail, dst_slab, dst_tail)
    return _out_call(acc, dinv)


# speedup vs baseline: 1.0175x; 1.0175x over previous
import jax
import jax.numpy as jnp
from jax import lax
from jax.experimental import pallas as pl
from jax.experimental.pallas import tpu as pltpu
from jax.experimental.pallas import tpu_sc as plsc

N_REAL = 10000
N_PAD = 10240
D = 128
NC, NS = 2, 16
NW = NC * NS
ROWS_PER_TILE = N_PAD // NS
E_PER_TILE = 10000
SLABS = 78
SLAB_W = 128
TAIL_W = E_PER_TILE - SLABS * SLAB_W
ROW_BLK = 2000


def _mesh():
    return plsc.VectorSubcoreMesh(core_axis_name="c", subcore_axis_name="s")


def _deg_body(dst_hbm, dtl_hbm, deg_out, idx_v, tidx_v, ones_v, zeros_v,
              deg_sh, sem):
    c = lax.axis_index("c")
    s = lax.axis_index("s")
    w = c * NS + s
    for i in range(SLAB_W // 16):
        ones_v[pl.ds(i * 16, 16)] = jnp.full((16,), 1.0, jnp.float32)
    for i in range(ROWS_PER_TILE // 16):
        zeros_v[pl.ds(i * 16, 16)] = jnp.zeros((16,), jnp.float32)
    pltpu.sync_copy(zeros_v, deg_sh.at[pl.ds(s * ROWS_PER_TILE, ROWS_PER_TILE)])
    pltpu.async_copy(dst_hbm.at[w], idx_v, sem).wait()
    pltpu.async_copy(dtl_hbm.at[w], tidx_v, sem).wait()
    plsc.subcore_barrier()

    @pl.loop(0, SLABS)
    def _(j):
        pltpu.sync_copy(ones_v, deg_sh.at[idx_v.at[j]], add=True)

    pltpu.sync_copy(ones_v.at[pl.ds(0, TAIL_W)], deg_sh.at[tidx_v.at[0]],
                    add=True)
    plsc.subcore_barrier()
    pltpu.sync_copy(deg_sh.at[pl.ds(s * ROWS_PER_TILE, ROWS_PER_TILE)],
                    deg_out.at[c, pl.ds(s * ROWS_PER_TILE, ROWS_PER_TILE)])


@jax.jit
def _deg_call(dst_slab, dst_tail):
    f = pl.kernel(
        _deg_body,
        out_type=jax.ShapeDtypeStruct((NC, N_PAD), jnp.float32),
        mesh=_mesh(),
        scratch_types=[
            pltpu.VMEM((SLABS, SLAB_W), jnp.int32),
            pltpu.VMEM((1, TAIL_W), jnp.int32),
            pltpu.VMEM((SLAB_W,), jnp.float32),
            pltpu.VMEM((ROWS_PER_TILE,), jnp.float32),
            pltpu.VMEM_SHARED((N_PAD,), jnp.float32),
            pltpu.SemaphoreType.DMA,
        ],
    )
    return f(dst_slab, dst_tail)


def _scale_body(x_ref, w_ref, deg_ref, y_ref, dinv_ref):
    d = deg_ref[0] + deg_ref[1] + 1.0
    dinv = lax.rsqrt(d)
    xw = jnp.dot(x_ref[...], w_ref[...], preferred_element_type=jnp.float32)
    y_ref[...] = xw * dinv
    dinv_ref[...] = dinv


@jax.jit
def _scale_call(x, W, deg3):
    return pl.pallas_call(
        _scale_body,
        grid=(N_REAL // ROW_BLK,),
        in_specs=[
            pl.BlockSpec((ROW_BLK, D), lambda i: (i, 0)),
            pl.BlockSpec((D, D), lambda i: (0, 0)),
            pl.BlockSpec((NC, ROW_BLK, 1), lambda i: (0, i, 0)),
        ],
        out_specs=[
            pl.BlockSpec((ROW_BLK, D), lambda i: (i, 0)),
            pl.BlockSpec((ROW_BLK, 1), lambda i: (i, 0)),
        ],
        out_shape=[
            jax.ShapeDtypeStruct((N_REAL, D), jnp.float32),
            jax.ShapeDtypeStruct((N_REAL, 1), jnp.float32),
        ],
    )(x, W, deg3)


def _msg_body(y_hbm, src_hbm, stl_hbm, dst_hbm, dtl_hbm, acc_out,
              src_v, stidx_v, dst_v, dtidx_v, buf0, acc_sh, sem, sem0):
    c = lax.axis_index("c")
    s = lax.axis_index("s")
    w = c * NS + s
    base = s * ROWS_PER_TILE

    @pl.loop(0, SLAB_W)
    def _(i):
        for k in range(D // 16):
            buf0[i, pl.ds(k * 16, 16)] = jnp.zeros((16,), jnp.float32)

    for t in range(ROWS_PER_TILE // SLAB_W):
        pltpu.sync_copy(buf0, acc_sh.at[pl.ds(base + t * SLAB_W, SLAB_W)])

    @pl.when(c == 0)
    def _():
        @pl.when(s < NS - 1)
        def _():
            pltpu.sync_copy(y_hbm.at[pl.ds(base, ROWS_PER_TILE)],
                            acc_sh.at[pl.ds(base, ROWS_PER_TILE)])

        @pl.when(s == NS - 1)
        def _():
            n_last = N_REAL - (NS - 1) * ROWS_PER_TILE
            pltpu.sync_copy(y_hbm.at[pl.ds(base, n_last)],
                            acc_sh.at[pl.ds(base, n_last)])

    pltpu.async_copy(src_hbm.at[w], src_v, sem).wait()
    pltpu.async_copy(stl_hbm.at[w], stidx_v, sem).wait()
    pltpu.async_copy(dst_hbm.at[w], dst_v, sem).wait()
    pltpu.async_copy(dtl_hbm.at[w], dtidx_v, sem).wait()
    plsc.subcore_barrier()

    @pl.loop(0, SLABS)
    def _(j):
        pltpu.async_copy(y_hbm.at[src_v.at[j]], buf0, sem0).wait()
        pltpu.sync_copy(buf0, acc_sh.at[dst_v.at[j]], add=True)

    pltpu.async_copy(y_hbm.at[stidx_v.at[0]], buf0.at[pl.ds(0, TAIL_W)],
                     sem0).wait()
    pltpu.sync_copy(buf0.at[pl.ds(0, TAIL_W)], acc_sh.at[dtidx_v.at[0]],
                    add=True)

    plsc.subcore_barrier()
    pltpu.sync_copy(acc_sh.at[pl.ds(base, ROWS_PER_TILE)],
                    acc_out.at[c, pl.ds(base, ROWS_PER_TILE)])


@jax.jit
def _msg_call(y, src_slab, src_tail, dst_slab, dst_tail):
    f = pl.kernel(
        _msg_body,
        out_type=jax.ShapeDtypeStruct((NC, N_PAD, D), jnp.float32),
        mesh=_mesh(),
        scratch_types=[
            pltpu.VMEM((SLABS, SLAB_W), jnp.int32),
            pltpu.VMEM((1, TAIL_W), jnp.int32),
            pltpu.VMEM((SLABS, SLAB_W), jnp.int32),
            pltpu.VMEM((1, TAIL_W), jnp.int32),
            pltpu.VMEM((SLAB_W, D), jnp.float32),
            pltpu.VMEM_SHARED((N_PAD, D), jnp.float32),
            pltpu.SemaphoreType.DMA,
            pltpu.SemaphoreType.DMA,
        ],
    )
    return f(y, src_slab, src_tail, dst_slab, dst_tail)


def _out_body(acc_ref, dinv_ref, o_ref):
    o_ref[...] = jnp.maximum((acc_ref[0] + acc_ref[1]) * dinv_ref[...], 0.0)


@jax.jit
def _out_call(acc, dinv):
    return pl.pallas_call(
        _out_body,
        grid=(N_REAL // ROW_BLK,),
        in_specs=[
            pl.BlockSpec((NC, ROW_BLK, D), lambda i: (0, i, 0)),
            pl.BlockSpec((ROW_BLK, 1), lambda i: (i, 0)),
        ],
        out_specs=pl.BlockSpec((ROW_BLK, D), lambda i: (i, 0)),
        out_shape=jax.ShapeDtypeStruct((N_REAL, D), jnp.float32),
    )(acc, dinv)


def kernel(x, edge_index, W):
    src = edge_index[0].astype(jnp.int32)
    dst = edge_index[1].astype(jnp.int32)
    n_main = NW * SLABS * SLAB_W
    src_slab = src[:n_main].reshape(NW, SLABS, SLAB_W)
    dst_slab = dst[:n_main].reshape(NW, SLABS, SLAB_W)
    src_tail = src[n_main:].reshape(NW, 1, TAIL_W)
    dst_tail = dst[n_main:].reshape(NW, 1, TAIL_W)

    deg_part = _deg_call(dst_slab, dst_tail)
    deg3 = deg_part.reshape(NC, N_PAD, 1)
    y, dinv = _scale_call(x, W, deg3)
    acc = _msg_call(y, src_slab, src_tail, dst_slab, dst_tail)
    return _out_call(acc, dinv)


# speedup vs baseline: 1.1110x; 1.0918x over previous
import jax
import jax.numpy as jnp
from jax import lax
from jax.experimental import pallas as pl
from jax.experimental.pallas import tpu as pltpu
from jax.experimental.pallas import tpu_sc as plsc

N_REAL = 10000
N_PAD = 10240
D = 128
NC, NS = 2, 16
NW = NC * NS
ROWS_PER_TILE = N_PAD // NS
SLABS = 79
SLAB_W = 128
ROW_BLK = 2000


def _mesh():
    return plsc.VectorSubcoreMesh(core_axis_name="c", subcore_axis_name="s")


def _deg_body(dst_hbm, deg_out, idx_v, ones_v, zeros_v, deg_sh, sem):
    c = lax.axis_index("c")
    s = lax.axis_index("s")
    w = c * NS + s
    for i in range(SLAB_W // 16):
        ones_v[pl.ds(i * 16, 16)] = jnp.full((16,), 1.0, jnp.float32)
    for i in range(ROWS_PER_TILE // 16):
        zeros_v[pl.ds(i * 16, 16)] = jnp.zeros((16,), jnp.float32)
    pltpu.sync_copy(zeros_v, deg_sh.at[pl.ds(s * ROWS_PER_TILE, ROWS_PER_TILE)])
    pltpu.async_copy(dst_hbm.at[w], idx_v, sem).wait()
    plsc.subcore_barrier()

    @pl.loop(0, SLABS)
    def _(j):
        pltpu.sync_copy(ones_v, deg_sh.at[idx_v.at[j]], add=True)

    plsc.subcore_barrier()
    pltpu.sync_copy(deg_sh.at[pl.ds(s * ROWS_PER_TILE, ROWS_PER_TILE)],
                    deg_out.at[c, pl.ds(s * ROWS_PER_TILE, ROWS_PER_TILE)])


@jax.jit
def _deg_call(dst_slab):
    f = pl.kernel(
        _deg_body,
        out_type=jax.ShapeDtypeStruct((NC, N_PAD), jnp.float32),
        mesh=_mesh(),
        scratch_types=[
            pltpu.VMEM((SLABS, SLAB_W), jnp.int32),
            pltpu.VMEM((SLAB_W,), jnp.float32),
            pltpu.VMEM((ROWS_PER_TILE,), jnp.float32),
            pltpu.VMEM_SHARED((N_PAD,), jnp.float32),
            pltpu.SemaphoreType.DMA,
        ],
    )
    return f(dst_slab)


def _scale_body(x_ref, w_ref, deg_ref, y_ref, dinv_ref):
    d = deg_ref[0] + deg_ref[1] + 1.0
    dinv = lax.rsqrt(d)
    xw = jnp.dot(x_ref[...], w_ref[...], preferred_element_type=jnp.float32)
    y_ref[...] = xw * dinv
    dinv_ref[...] = dinv


@jax.jit
def _scale_call(x, W, deg3):
    return pl.pallas_call(
        _scale_body,
        grid=(N_REAL // ROW_BLK,),
        in_specs=[
            pl.BlockSpec((ROW_BLK, D), lambda i: (i, 0)),
            pl.BlockSpec((D, D), lambda i: (0, 0)),
            pl.BlockSpec((NC, ROW_BLK, 1), lambda i: (0, i, 0)),
        ],
        out_specs=[
            pl.BlockSpec((ROW_BLK, D), lambda i: (i, 0)),
            pl.BlockSpec((ROW_BLK, 1), lambda i: (i, 0)),
        ],
        out_shape=[
            jax.ShapeDtypeStruct((N_REAL, D), jnp.float32),
            jax.ShapeDtypeStruct((N_REAL, 1), jnp.float32),
        ],
    )(x, W, deg3)


def _msg_body(y_hbm, src_hbm, dst_hbm, acc_out,
              src_v, dst_v, buf0, acc_sh, sem, sem0):
    c = lax.axis_index("c")
    s = lax.axis_index("s")
    w = c * NS + s
    base = s * ROWS_PER_TILE

    @pl.loop(0, SLAB_W)
    def _(i):
        for k in range(D // 16):
            buf0[i, pl.ds(k * 16, 16)] = jnp.zeros((16,), jnp.float32)

    for t in range(ROWS_PER_TILE // SLAB_W):
        pltpu.sync_copy(buf0, acc_sh.at[pl.ds(base + t * SLAB_W, SLAB_W)])

    @pl.when(c == 0)
    def _():
        @pl.when(s < NS - 1)
        def _():
            pltpu.sync_copy(y_hbm.at[pl.ds(base, ROWS_PER_TILE)],
                            acc_sh.at[pl.ds(base, ROWS_PER_TILE)])

        @pl.when(s == NS - 1)
        def _():
            n_last = N_REAL - (NS - 1) * ROWS_PER_TILE
            pltpu.sync_copy(y_hbm.at[pl.ds(base, n_last)],
                            acc_sh.at[pl.ds(base, n_last)])

    pltpu.async_copy(src_hbm.at[w], src_v, sem).wait()
    pltpu.async_copy(dst_hbm.at[w], dst_v, sem).wait()
    plsc.subcore_barrier()

    @pl.loop(0, SLABS)
    def _(j):
        pltpu.async_copy(y_hbm.at[src_v.at[j]], buf0, sem0).wait()
        pltpu.sync_copy(buf0, acc_sh.at[dst_v.at[j]], add=True)

    plsc.subcore_barrier()
    pltpu.sync_copy(acc_sh.at[pl.ds(base, ROWS_PER_TILE)],
                    acc_out.at[c, pl.ds(base, ROWS_PER_TILE)])


@jax.jit
def _msg_call(y, src_slab, dst_slab):
    f = pl.kernel(
        _msg_body,
        out_type=jax.ShapeDtypeStruct((NC, N_PAD, D), jnp.float32),
        mesh=_mesh(),
        scratch_types=[
            pltpu.VMEM((SLABS, SLAB_W), jnp.int32),
            pltpu.VMEM((SLABS, SLAB_W), jnp.int32),
            pltpu.VMEM((SLAB_W, D), jnp.float32),
            pltpu.VMEM_SHARED((N_PAD, D), jnp.float32),
            pltpu.SemaphoreType.DMA,
            pltpu.SemaphoreType.DMA,
        ],
    )
    return f(y, src_slab, dst_slab)


def _out_body(acc_ref, dinv_ref, o_ref):
    o_ref[...] = jnp.maximum((acc_ref[0] + acc_ref[1]) * dinv_ref[...], 0.0)


@jax.jit
def _out_call(acc, dinv):
    return pl.pallas_call(
        _out_body,
        grid=(N_REAL // ROW_BLK,),
        in_specs=[
            pl.BlockSpec((NC, ROW_BLK, D), lambda i: (0, i, 0)),
            pl.BlockSpec((ROW_BLK, 1), lambda i: (i, 0)),
        ],
        out_specs=pl.BlockSpec((ROW_BLK, D), lambda i: (i, 0)),
        out_shape=jax.ShapeDtypeStruct((N_REAL, D), jnp.float32),
    )(acc, dinv)


def kernel(x, edge_index, W):
    N = x.shape[0]
    src = edge_index[0].astype(jnp.int32)
    dst = edge_index[1].astype(jnp.int32)
    pad_n = NW * SLABS * SLAB_W - src.shape[0]
    pad_ar = jnp.arange(pad_n, dtype=jnp.int32)
    src_all = jnp.concatenate([src, pad_ar % N])
    dst_all = jnp.concatenate([dst, N + pad_ar % (N_PAD - N)])
    src_slab = src_all.reshape(NW, SLABS, SLAB_W)
    dst_slab = dst_all.reshape(NW, SLABS, SLAB_W)

    deg_part = _deg_call(dst_slab)
    deg3 = deg_part.reshape(NC, N_PAD, 1)
    y, dinv = _scale_call(x, W, deg3)
    acc = _msg_call(y, src_slab, dst_slab)
    return _out_call(acc, dinv)
